# baseline (speedup 1.0000x reference)
import jax
import jax.numpy as jnp
from jax import lax
from jax.experimental import pallas as pl
from jax.experimental.pallas import tpu as pltpu
from jax.experimental.pallas import tpu_sc as plsc

VOCAB = 100000
HIDDEN = 1024
BATCH = 4
SEQ = 2048

TOKENS = BATCH * SEQ
NW = 32
P_PER_W = SEQ // NW
CH = 8
NPC = P_PER_W // CH
NSTEP = NPC * BATCH
NBUF = 12
AHEAD = 6
LANES = 16
VPR = HIDDEN // LANES


def _emb_body(x_hbm, pos_hbm, table_hbm, out_hbm,
              idx_v, bufarr, posarr, gsem_arr, wsem_arr, psem, isem):
    nc = 2
    wid = lax.axis_index("s") * nc + lax.axis_index("c")
    p0 = wid * P_PER_W

    bufs = tuple(bufarr.at[i] for i in range(NBUF))
    gsems = tuple(gsem_arr.at[i] for i in range(NBUF))
    wsems = tuple(wsem_arr.at[i] for i in range(NBUF))
    posbs = tuple(posarr.at[i] for i in range(2))

    idx_copies = []
    for b in range(BATCH):
        idx_copies.append(pltpu.make_async_copy(
            x_hbm.at[pl.ds(b * SEQ + p0, P_PER_W)],
            idx_v.at[pl.ds(b * P_PER_W, P_PER_W)], isem))
        idx_copies[-1].start()
    for cp in idx_copies:
        cp.wait()

    def gsrc(j):
        pc, b = divmod(j, BATCH)
        return table_hbm.at[idx_v.at[pl.ds(b * P_PER_W + pc * CH, CH)]]

    def odst(j):
        pc, b = divmod(j, BATCH)
        return out_hbm.at[pl.ds(b * SEQ + p0 + pc * CH, CH)]

    def fire_pos(pc):
        pltpu.async_copy(pos_hbm.at[pl.ds(p0 + pc * CH, CH)],
                         posbs[pc % 2], psem)

    fire_pos(0)
    for j in range(AHEAD):
        pltpu.async_copy(gsrc(j), bufs[j % NBUF], gsems[j % NBUF])

    for j in range(NSTEP):
        pc, b = divmod(j, BATCH)
        cur = bufs[j % NBUF]

        if j + AHEAD < NSTEP:
            k = j + AHEAD
            if k >= NBUF:
                pltpu.make_async_copy(bufs[k % NBUF], odst(k - NBUF),
                                      wsems[k % NBUF]).wait()
            pltpu.async_copy(gsrc(k), bufs[k % NBUF], gsems[k % NBUF])

        if b == 0:
            pltpu.make_async_copy(pos_hbm.at[pl.ds(p0 + pc * CH, CH)],
                                  posbs[pc % 2], psem).wait()

        pltpu.make_async_copy(gsrc(j), cur, gsems[j % NBUF]).wait()

        posb = posbs[pc % 2]

        @plsc.parallel_loop(0, CH * VPR, unroll=8)
        def _(i):
            r = i >> 6
            col = (i & (VPR - 1)) * LANES
            plsc.addupdate(cur.at[r, pl.ds(col, LANES)],
                           posb[r, pl.ds(col, LANES)])

        pltpu.async_copy(cur, odst(j), wsems[j % NBUF])

        if b == BATCH - 1 and pc + 1 < NPC:
            fire_pos(pc + 1)

    for j in range(NSTEP - NBUF, NSTEP):
        pltpu.make_async_copy(bufs[j % NBUF], odst(j),
                              wsems[j % NBUF]).wait()


@jax.jit
def kernel(x, token_table, pos_emb):
    pos = pos_emb.reshape(SEQ, HIDDEN)
    mesh = plsc.VectorSubcoreMesh(core_axis_name="c", subcore_axis_name="s",
                                  num_cores=2, num_subcores=16)
    out = pl.kernel(
        _emb_body,
        out_type=jax.ShapeDtypeStruct((TOKENS, HIDDEN), jnp.float32),
        mesh=mesh,
        scratch_types=[
            pltpu.VMEM((BATCH * P_PER_W,), jnp.int32),
            pltpu.VMEM((NBUF, CH, HIDDEN), jnp.float32),
            pltpu.VMEM((2, CH, HIDDEN), jnp.float32),
            pltpu.SemaphoreType.DMA((NBUF,)),
            pltpu.SemaphoreType.DMA((NBUF,)),
            pltpu.SemaphoreType.DMA,
            pltpu.SemaphoreType.DMA,
        ],
    )(x.reshape(TOKENS).astype(jnp.int32), pos, token_table)
    return out.reshape(BATCH, SEQ, HIDDEN)


# speedup vs baseline: 1.0710x; 1.0710x over previous
import jax
import jax.numpy as jnp
from jax import lax
from jax.experimental import pallas as pl
from jax.experimental.pallas import tpu as pltpu
from jax.experimental.pallas import tpu_sc as plsc

VOCAB = 100000
HIDDEN = 1024
BATCH = 4
SEQ = 2048

TOKENS = BATCH * SEQ
NW = 32
P_PER_W = SEQ // NW
CH = 16
NPC = P_PER_W // CH
NSTEP = NPC * BATCH
NBUF = 5
AHEAD = 3
LANES = 16
VPR = HIDDEN // LANES


def _emb_body(x_hbm, pos_hbm, table_hbm, out_hbm,
              idx_v, buf0, buf1, buf2, buf3, buf4, posb0, posb1,
              g0, g1, g2, g3, g4, w0, w1, w2, w3, w4, psem,
              i0, i1, i2, i3):
    nc = 2
    wid = lax.axis_index("s") * nc + lax.axis_index("c")
    p0 = wid * P_PER_W

    bufs = (buf0, buf1, buf2, buf3, buf4)
    gsems = (g0, g1, g2, g3, g4)
    wsems = (w0, w1, w2, w3, w4)
    posbs = (posb0, posb1)

    isems = (i0, i1, i2, i3)
    idx_copies = []
    for b in range(BATCH):
        idx_copies.append(pltpu.make_async_copy(
            x_hbm.at[pl.ds(b * SEQ + p0, P_PER_W)],
            idx_v.at[pl.ds(b * P_PER_W, P_PER_W)], isems[b]))
        idx_copies[-1].start()

    def gsrc(j):
        pc, b = divmod(j, BATCH)
        return table_hbm.at[idx_v.at[pl.ds(b * P_PER_W + pc * CH, CH)]]

    def odst(j):
        pc, b = divmod(j, BATCH)
        return out_hbm.at[pl.ds(b * SEQ + p0 + pc * CH, CH)]

    def fire_pos(pc):
        pltpu.async_copy(pos_hbm.at[pl.ds(p0 + pc * CH, CH)],
                         posbs[pc % 2], psem)

    fire_pos(0)
    for j in range(AHEAD):
        idx_copies[j].wait()
        pltpu.async_copy(gsrc(j), bufs[j % NBUF], gsems[j % NBUF])
    for b in range(AHEAD, BATCH):
        idx_copies[b].wait()

    for j in range(NSTEP):
        pc, b = divmod(j, BATCH)
        cur = bufs[j % NBUF]

        if j + AHEAD < NSTEP:
            k = j + AHEAD
            if k >= NBUF:
                pltpu.make_async_copy(bufs[k % NBUF], odst(k - NBUF),
                                      wsems[k % NBUF]).wait()
            pltpu.async_copy(gsrc(k), bufs[k % NBUF], gsems[k % NBUF])

        if b == 0:
            pltpu.make_async_copy(pos_hbm.at[pl.ds(p0 + pc * CH, CH)],
                                  posbs[pc % 2], psem).wait()

        pltpu.make_async_copy(gsrc(j), cur, gsems[j % NBUF]).wait()

        posb = posbs[pc % 2]

        @plsc.parallel_loop(0, CH * VPR, unroll=8)
        def _(i):
            r = i >> 6
            col = (i & (VPR - 1)) * LANES
            plsc.addupdate(cur.at[r, pl.ds(col, LANES)],
                           posb[r, pl.ds(col, LANES)])

        pltpu.async_copy(cur, odst(j), wsems[j % NBUF])

        if b == BATCH - 1 and pc + 1 < NPC:
            fire_pos(pc + 1)

    for j in range(NSTEP - NBUF, NSTEP):
        pltpu.make_async_copy(bufs[j % NBUF], odst(j),
                              wsems[j % NBUF]).wait()


@jax.jit
def kernel(x, token_table, pos_emb):
    pos = pos_emb.reshape(SEQ, HIDDEN)
    mesh = plsc.VectorSubcoreMesh(core_axis_name="c", subcore_axis_name="s",
                                  num_cores=2, num_subcores=16)
    out = pl.kernel(
        _emb_body,
        out_type=jax.ShapeDtypeStruct((TOKENS, HIDDEN), jnp.float32),
        mesh=mesh,
        scratch_types=[
            pltpu.VMEM((BATCH * P_PER_W,), jnp.int32),
            pltpu.VMEM((CH, HIDDEN), jnp.float32),
            pltpu.VMEM((CH, HIDDEN), jnp.float32),
            pltpu.VMEM((CH, HIDDEN), jnp.float32),
            pltpu.VMEM((CH, HIDDEN), jnp.float32),
            pltpu.VMEM((CH, HIDDEN), jnp.float32),
            pltpu.VMEM((CH, HIDDEN), jnp.float32),
            pltpu.VMEM((CH, HIDDEN), jnp.float32),
            pltpu.SemaphoreType.DMA,
            pltpu.SemaphoreType.DMA,
            pltpu.SemaphoreType.DMA,
            pltpu.SemaphoreType.DMA,
            pltpu.SemaphoreType.DMA,
            pltpu.SemaphoreType.DMA,
            pltpu.SemaphoreType.DMA,
            pltpu.SemaphoreType.DMA,
            pltpu.SemaphoreType.DMA,
            pltpu.SemaphoreType.DMA,
            pltpu.SemaphoreType.DMA,
            pltpu.SemaphoreType.DMA,
            pltpu.SemaphoreType.DMA,
            pltpu.SemaphoreType.DMA,
            pltpu.SemaphoreType.DMA,
        ],
    )(x.reshape(TOKENS).astype(jnp.int32), pos, token_table)
    return out.reshape(BATCH, SEQ, HIDDEN)
